# Initial kernel scaffold; baseline (speedup 1.0000x reference)
import jax
import jax.numpy as jnp
from jax.experimental import pallas as pl


def kernel(delivery_area_id, order_hour, order_weekday, basket_avg_eur, orders_cnt_log, area_table, hour_table, weekday_table):
    raise NotImplementedError("write your pallas kernel here")


# speedup vs baseline: 1.6180x; 1.6180x over previous
import functools

import jax
import jax.numpy as jnp
from jax import lax
from jax.experimental import pallas as pl
from jax.experimental.pallas import tpu as pltpu
from jax.experimental.pallas import tpu_sc as plsc

BATCH = 16384
EMB = 16
OUT_W = 50
NUM_WORKERS = 32
BPW = BATCH // NUM_WORKERS
CHUNK = 128
NCHUNK = BPW // CHUNK


def _make_kernel():
    mesh = plsc.VectorSubcoreMesh(core_axis_name="c", subcore_axis_name="s")

    @functools.partial(
        pl.kernel,
        out_type=jax.ShapeDtypeStruct((BATCH * OUT_W,), jnp.float32),
        mesh=mesh,
        compiler_params=pltpu.CompilerParams(use_tc_tiling_on_sc=False),
        scratch_types=[
            pltpu.VMEM((NCHUNK, CHUNK), jnp.int32),
            pltpu.VMEM((NCHUNK, CHUNK), jnp.int32),
            pltpu.VMEM((NCHUNK, CHUNK), jnp.int32),
            pltpu.VMEM((NCHUNK, CHUNK, EMB), jnp.float32),
            pltpu.VMEM((NCHUNK, CHUNK, EMB), jnp.float32),
            pltpu.VMEM((NCHUNK, CHUNK, EMB), jnp.float32),
            pltpu.VMEM((2 * BPW + 16,), jnp.float32),
            pltpu.VMEM((BPW * OUT_W + 16,), jnp.float32),
            pltpu.SemaphoreType.DMA,
            pltpu.SemaphoreType.DMA,
            pltpu.SemaphoreType.DMA,
        ],
    )
    def k(aid_hbm, hr_hbm, wd_hbm, nm_hbm, at_hbm, ht_hbm, wt_hbm,
          out_hbm, aidx, hidx, widx, ra, rh, rw, nm, ob, sa, sh, sw):
        wid = lax.axis_index("s") * 2 + lax.axis_index("c")
        base = wid * BPW

        pltpu.sync_copy(aid_hbm.at[pl.ds(wid * NCHUNK, NCHUNK)], aidx)
        pltpu.sync_copy(hr_hbm.at[pl.ds(wid * NCHUNK, NCHUNK)], hidx)
        pltpu.sync_copy(wd_hbm.at[pl.ds(wid * NCHUNK, NCHUNK)], widx)

        def shift(i, carry):
            for j in range(NCHUNK):
                aidx[j, pl.ds(i * 16, 16)] = aidx[j, pl.ds(i * 16, 16)] + 1
                hidx[j, pl.ds(i * 16, 16)] = hidx[j, pl.ds(i * 16, 16)] + 1
                widx[j, pl.ds(i * 16, 16)] = widx[j, pl.ds(i * 16, 16)] + 1
            return carry
        lax.fori_loop(0, CHUNK // 16, shift, 0)

        copies = []
        for j in range(NCHUNK):
            copies.append(pltpu.async_copy(at_hbm.at[aidx.at[j]], ra.at[j], sa))
            copies.append(pltpu.async_copy(ht_hbm.at[hidx.at[j]], rh.at[j], sh))
            copies.append(pltpu.async_copy(wt_hbm.at[widx.at[j]], rw.at[j], sw))
        pltpu.sync_copy(nm_hbm.at[pl.ds(base * 2, 2 * BPW)],
                        nm.at[pl.ds(0, 2 * BPW)])
        for c in copies:
            c.wait()

        for j in range(NCHUNK):
            def row(r, carry, j=j):
                b = j * CHUNK + r
                o = b * OUT_W
                ob[pl.ds(o, EMB)] = ra[j, r, :]
                ob[pl.ds(o + EMB, EMB)] = rh[j, r, :]
                ob[pl.ds(o + 2 * EMB, EMB)] = rw[j, r, :]
                ob[pl.ds(o + 3 * EMB, 16)] = nm[pl.ds(2 * b, 16)]
                return carry
            lax.fori_loop(0, CHUNK, row, 0)

        pltpu.sync_copy(ob.at[pl.ds(0, BPW * OUT_W)],
                        out_hbm.at[pl.ds(base * OUT_W, BPW * OUT_W)])

    return k


_sc_kernel = None


def kernel(delivery_area_id, order_hour, order_weekday, basket_avg_eur,
           orders_cnt_log, area_table, hour_table, weekday_table):
    global _sc_kernel
    if _sc_kernel is None:
        _sc_kernel = _make_kernel()
    aid = jnp.reshape(delivery_area_id.astype(jnp.int32),
                      (NUM_WORKERS * NCHUNK, CHUNK))
    hr = jnp.reshape(order_hour.astype(jnp.int32), (NUM_WORKERS * NCHUNK, CHUNK))
    wd = jnp.reshape(order_weekday.astype(jnp.int32),
                     (NUM_WORKERS * NCHUNK, CHUNK))
    nm = jnp.stack([jnp.reshape(basket_avg_eur.astype(jnp.float32), (BATCH,)),
                    jnp.reshape(orders_cnt_log.astype(jnp.float32), (BATCH,))],
                   axis=1).reshape(2 * BATCH)
    flat = _sc_kernel(aid, hr, wd, nm,
                      area_table.astype(jnp.float32),
                      hour_table.astype(jnp.float32),
                      weekday_table.astype(jnp.float32))
    return jnp.reshape(flat, (BATCH, OUT_W))


# speedup vs baseline: 1.6714x; 1.0330x over previous
import functools

import jax
import jax.numpy as jnp
from jax import lax
from jax.experimental import pallas as pl
from jax.experimental.pallas import tpu as pltpu
from jax.experimental.pallas import tpu_sc as plsc

BATCH = 16384
EMB = 16
OUT_W = 50
NUM_WORKERS = 32
BPW = BATCH // NUM_WORKERS
CHUNK = 128
NCHUNK = BPW // CHUNK


def _make_kernel():
    mesh = plsc.VectorSubcoreMesh(core_axis_name="c", subcore_axis_name="s")

    @functools.partial(
        pl.kernel,
        out_type=jax.ShapeDtypeStruct((BATCH, OUT_W), jnp.float32),
        mesh=mesh,
        compiler_params=pltpu.CompilerParams(use_tc_tiling_on_sc=False),
        scratch_types=[
            pltpu.VMEM((NCHUNK, CHUNK), jnp.int32),
            pltpu.VMEM((NCHUNK, CHUNK), jnp.int32),
            pltpu.VMEM((NCHUNK, CHUNK), jnp.int32),
            pltpu.VMEM((NCHUNK * CHUNK, EMB), jnp.float32),
            pltpu.VMEM((NCHUNK * CHUNK, EMB), jnp.float32),
            pltpu.VMEM((NCHUNK * CHUNK, EMB), jnp.float32),
            pltpu.VMEM((BPW, 2), jnp.float32),
            pltpu.SemaphoreType.DMA,
            pltpu.SemaphoreType.DMA,
            pltpu.SemaphoreType.DMA,
            pltpu.SemaphoreType.DMA,
        ],
    )
    def k(aid_hbm, hr_hbm, wd_hbm, nm_hbm, at_hbm, ht_hbm, wt_hbm,
          out_hbm, aidx, hidx, widx, ra, rh, rw, nm, sa, sh, sw, so):
        wid = lax.axis_index("s") * 2 + lax.axis_index("c")
        base = wid * BPW

        pltpu.sync_copy(aid_hbm.at[pl.ds(wid * NCHUNK, NCHUNK)], aidx)
        pltpu.sync_copy(hr_hbm.at[pl.ds(wid * NCHUNK, NCHUNK)], hidx)
        pltpu.sync_copy(wd_hbm.at[pl.ds(wid * NCHUNK, NCHUNK)], widx)

        def shift(i, carry):
            for j in range(NCHUNK):
                aidx[j, pl.ds(i * 16, 16)] = aidx[j, pl.ds(i * 16, 16)] + 1
                hidx[j, pl.ds(i * 16, 16)] = hidx[j, pl.ds(i * 16, 16)] + 1
                widx[j, pl.ds(i * 16, 16)] = widx[j, pl.ds(i * 16, 16)] + 1
            return carry
        lax.fori_loop(0, CHUNK // 16, shift, 0)

        copies = []
        for j in range(NCHUNK):
            copies.append(pltpu.async_copy(
                at_hbm.at[aidx.at[j]], ra.at[pl.ds(j * CHUNK, CHUNK)], sa))
            copies.append(pltpu.async_copy(
                ht_hbm.at[hidx.at[j]], rh.at[pl.ds(j * CHUNK, CHUNK)], sh))
            copies.append(pltpu.async_copy(
                wt_hbm.at[widx.at[j]], rw.at[pl.ds(j * CHUNK, CHUNK)], sw))
        nmc = pltpu.async_copy(nm_hbm.at[pl.ds(base, BPW)], nm, so)
        for c in copies:
            c.wait()
        nmc.wait()

        w0 = pltpu.async_copy(
            ra, out_hbm.at[pl.ds(base, BPW), pl.ds(0, EMB)], sa)
        w1 = pltpu.async_copy(
            rh, out_hbm.at[pl.ds(base, BPW), pl.ds(EMB, EMB)], sh)
        w2 = pltpu.async_copy(
            rw, out_hbm.at[pl.ds(base, BPW), pl.ds(2 * EMB, EMB)], sw)
        w3 = pltpu.async_copy(
            nm, out_hbm.at[pl.ds(base, BPW), pl.ds(3 * EMB, 2)], so)
        w0.wait()
        w1.wait()
        w2.wait()
        w3.wait()

    return k


_sc_kernel = None


def kernel(delivery_area_id, order_hour, order_weekday, basket_avg_eur,
           orders_cnt_log, area_table, hour_table, weekday_table):
    global _sc_kernel
    if _sc_kernel is None:
        _sc_kernel = _make_kernel()
    aid = jnp.reshape(delivery_area_id.astype(jnp.int32),
                      (NUM_WORKERS * NCHUNK, CHUNK))
    hr = jnp.reshape(order_hour.astype(jnp.int32), (NUM_WORKERS * NCHUNK, CHUNK))
    wd = jnp.reshape(order_weekday.astype(jnp.int32),
                     (NUM_WORKERS * NCHUNK, CHUNK))
    nm = jnp.stack([jnp.reshape(basket_avg_eur.astype(jnp.float32), (BATCH,)),
                    jnp.reshape(orders_cnt_log.astype(jnp.float32), (BATCH,))],
                   axis=1)
    return _sc_kernel(aid, hr, wd, nm,
                      area_table.astype(jnp.float32),
                      hour_table.astype(jnp.float32),
                      weekday_table.astype(jnp.float32))
